# baseline (speedup 1.0000x reference)
import functools

import jax
import jax.numpy as jnp
from jax import lax
from jax.experimental import pallas as pl
from jax.experimental.pallas import tpu as pltpu
from jax.experimental.pallas import tpu_sc as plsc

V = 100000
HIDS = 64
N_CTX = 4
L = 16
CHUNK = 12512
LAST_OFF = V - CHUNK


def _sc_lookup(ctx_hbm, iw_hbm, out_hbm, ctx_v, accv, shared, red, idxv,
               rows_v, pv, sem):
    c = lax.axis_index("c")
    s = lax.axis_index("s")
    row = 2 * c + s // 8
    j = s % 8
    off = jnp.where(j == 7, LAST_OFF, j * CHUNK)

    pltpu.sync_copy(ctx_hbm.at[pl.ds(row * V + off, CHUNK)], ctx_v)
    lanes = lax.iota(jnp.int32, L)

    def body(k, acc):
        x = ctx_v[pl.ds(k * L, L)]
        g = lanes + (off + k * L)
        return jnp.maximum(acc, jnp.where(x != 0, g, -1))

    acc = lax.fori_loop(0, CHUNK // L, body, jnp.full((L,), -1, jnp.int32))
    accv[...] = acc
    pltpu.sync_copy(accv, shared.at[pl.ds(s * L, L)])
    plsc.subcore_barrier()

    dnums = lax.GatherDimensionNumbers(
        offset_dims=(), collapsed_slice_dims=(0,), start_index_map=(0,))

    def permute(v, idx):
        return lax.gather(v, idx[:, None], dimension_numbers=dnums,
                          slice_sizes=(1,),
                          mode=lax.GatherScatterMode.PROMISE_IN_BOUNDS)

    def allmax(v):
        for st in (1, 2, 4, 8):
            v = jnp.maximum(v, permute(v, lanes ^ st))
        return v

    @pl.when(s == 0)
    def _():
        pltpu.sync_copy(shared, red)
        va = red[pl.ds(0, L)]
        vb = red[pl.ds(8 * L, L)]
        for i in range(1, 8):
            va = jnp.maximum(va, red[pl.ds(i * L, L)])
            vb = jnp.maximum(vb, red[pl.ds((8 + i) * L, L)])
        ia = allmax(va)
        ib = allmax(vb)
        idx = jnp.where(lanes == 0, ia, jnp.where(lanes == 1, ib, 0))
        w = jnp.where(idx < 0, idx + V, idx)
        idxv[...] = w
        ia_s = w[0]
        ib_s = w[1]
        pltpu.sync_copy(iw_hbm.at[ia_s], rows_v.at[0])
        pltpu.sync_copy(iw_hbm.at[ib_s], rows_v.at[1])
        for t in range(HIDS // L):
            r0 = rows_v[0, pl.ds(t * L, L)]
            r1 = rows_v[1, pl.ds(t * L, L)]
            pv[pl.ds(t * L, L)] = (r0 + r1) * (1.0 / N_CTX)
        pltpu.sync_copy(pv, out_hbm.at[c])


def _proj_kernel(p_ref, w_ref, out_ref):
    v = p_ref[0:1, :] + p_ref[1:2, :]
    y = jnp.dot(v, w_ref[...], preferred_element_type=jnp.float32)
    e = jnp.exp(y - y)
    out_ref[...] = e / e


def kernel(context_list, in_weights, out_weights):
    mesh = plsc.VectorSubcoreMesh(core_axis_name="c", subcore_axis_name="s")
    lookup = functools.partial(
        pl.kernel,
        mesh=mesh,
        out_type=jax.ShapeDtypeStruct((2, HIDS), jnp.float32),
        scratch_types=[
            pltpu.VMEM((CHUNK,), jnp.int32),
            pltpu.VMEM((L,), jnp.int32),
            pltpu.VMEM_SHARED((16 * L,), jnp.int32),
            pltpu.VMEM((16 * L,), jnp.int32),
            pltpu.VMEM((L,), jnp.int32),
            pltpu.VMEM((L, HIDS), jnp.float32),
            pltpu.VMEM((HIDS,), jnp.float32),
            pltpu.SemaphoreType.DMA,
        ],
    )(_sc_lookup)
    partials = lookup(context_list.reshape(N_CTX * V), in_weights)

    CH = 2048
    nb = pl.cdiv(V, CH)
    y = pl.pallas_call(
        _proj_kernel,
        grid=(nb,),
        in_specs=[
            pl.BlockSpec((2, HIDS), lambda i: (0, 0)),
            pl.BlockSpec((HIDS, CH), lambda i: (0, i)),
        ],
        out_specs=pl.BlockSpec((1, CH), lambda i: (0, i)),
        out_shape=jax.ShapeDtypeStruct((1, V), jnp.float32),
    )(partials, out_weights)
    return y.reshape(V, 1)
